# baseline (speedup 1.0000x reference)
import functools

import jax
import jax.numpy as jnp
from jax import lax
from jax.experimental import pallas as pl
from jax.experimental.pallas import tpu as pltpu
from jax.experimental.pallas import tpu_sc as plsc

N_NODES = 10000
N_FEAT = 128
N_EDGES = 320000

NUM_CORES = 2
NUM_SUBCORES = 16
NUM_WORKERS = NUM_CORES * NUM_SUBCORES
EDGES_PER_WORKER = N_EDGES // NUM_WORKERS
CHUNK = 48
GROUPS = CHUNK // 16
N_CHUNKS = -(-EDGES_PER_WORKER // CHUNK)
N_PAIRS = N_CHUNKS // 2
LAST_BASE = EDGES_PER_WORKER - CHUNK


def _scale_body(h_ref, r_ref, o_ref):
    o_ref[...] = h_ref[...] * r_ref[...]


def _scale_h_by_r(h, r):
    return pl.pallas_call(
        _scale_body,
        out_shape=jax.ShapeDtypeStruct((N_NODES, N_FEAT), jnp.float32),
    )(h, r.reshape(1, N_FEAT))


_mesh = plsc.VectorSubcoreMesh(core_axis_name="c", subcore_axis_name="s")


@functools.partial(
    pl.kernel,
    mesh=_mesh,
    out_type=jax.ShapeDtypeStruct((N_EDGES,), jnp.float32),
    scratch_types=[
        pltpu.VMEM((EDGES_PER_WORKER,), jnp.int32),
        pltpu.VMEM((EDGES_PER_WORKER,), jnp.int32),
        pltpu.VMEM((CHUNK, N_FEAT), jnp.float32),
        pltpu.VMEM((CHUNK, N_FEAT), jnp.float32),
        pltpu.VMEM((CHUNK, N_FEAT), jnp.float32),
        pltpu.VMEM((CHUNK, N_FEAT), jnp.float32),
        pltpu.VMEM((CHUNK,), jnp.float32),
        pltpu.VMEM((CHUNK,), jnp.float32),
        pltpu.SemaphoreType.DMA,
        pltpu.SemaphoreType.DMA,
        pltpu.SemaphoreType.DMA,
        pltpu.SemaphoreType.DMA,
        pltpu.SemaphoreType.DMA,
        pltpu.SemaphoreType.DMA,
        pltpu.VMEM_SHARED((N_NODES, N_FEAT), jnp.float32),
    ],
)
def _edge_scores(hr_hbm, h_hbm, ei_hbm, out_hbm,
                 idx_u, idx_v, u0, u1, v0, v1, out0, out1,
                 sem_u0, sem_u1, sem_v0, sem_v1, sem_o0, sem_o1, hr_sp):
    sid = lax.axis_index("s")
    wid = sid * NUM_CORES + lax.axis_index("c")
    base0 = wid * EDGES_PER_WORKER
    lanes = lax.iota(jnp.int32, 16)

    @pl.when(sid < 15)
    def _():
        pltpu.sync_copy(hr_hbm.at[pl.ds(sid * 640, 640), :],
                        hr_sp.at[pl.ds(sid * 640, 640), :])

    @pl.when(sid == 15)
    def _():
        pltpu.sync_copy(hr_hbm.at[pl.ds(9600, 400), :],
                        hr_sp.at[pl.ds(9600, 400), :])

    pltpu.sync_copy(ei_hbm.at[pl.ds(base0, EDGES_PER_WORKER)], idx_u)
    pltpu.sync_copy(ei_hbm.at[pl.ds(N_EDGES + base0, EDGES_PER_WORKER)], idx_v)
    plsc.subcore_barrier()

    def gathers(base, u_buf, v_buf, su, sv):
        half = CHUNK // 2
        cps = []
        for s, (tab, idx, buf) in ((su, (hr_hbm, idx_u, u_buf)),
                                   (sv, (h_hbm, idx_v, v_buf))):
            for hoff in (0, half):
                cps.append(pltpu.async_copy(
                    tab.at[idx.at[pl.ds(base + hoff, half)]],
                    buf.at[pl.ds(hoff, half), :], s))
        return cps

    msk8 = lanes < 8
    msk4 = (lanes & 4) == 0
    quad_pat = (lanes & 1) * 8 + ((lanes >> 1) & 1) * 4

    def step(v, sh):
        return v + jnp.take(v, lanes ^ sh)

    def compute(u_buf, v_buf, out_buf):
        def quad_body(k, scores):
            q = k & 3
            accs = []
            for t in range(4):
                e = k * 4 + t
                p = [u_buf[e, pl.ds(16 * j, 16)] * v_buf[e, pl.ds(16 * j, 16)]
                     for j in range(N_FEAT // 16)]
                accs.append(((p[0] + p[1]) + (p[2] + p[3]))
                            + ((p[4] + p[5]) + (p[6] + p[7])))
            m0 = jnp.where(msk8, step(accs[0], 8), step(accs[1], 8))
            m1 = jnp.where(msk8, step(accs[2], 8), step(accs[3], 8))
            n = jnp.where(msk4, step(m0, 4), step(m1, 4))
            full = step(step(n, 2), 1)
            scores = jnp.where(q == 0, jnp.zeros((16,), jnp.float32), scores)
            scores = jnp.where((lanes >> 2) == q, jnp.take(full, quad_pat), scores)
            out_buf[pl.ds((k >> 2) * 16, 16)] = scores
            return scores

        lax.fori_loop(0, CHUNK // 4, quad_body, jnp.zeros((16,), jnp.float32))

    def writeback(base, out_buf, sem):
        return pltpu.async_copy(out_buf, out_hbm.at[pl.ds(base0 + base, CHUNK)], sem)

    def reclaim(out_buf, sem):
        pltpu.make_async_copy(out_buf, out_hbm.at[pl.ds(base0, CHUNK)], sem).wait()

    for cp in gathers(0, u0, v0, sem_u0, sem_v0):
        cp.wait()

    def pair_body(i, carry):
        c0_base = (2 * i) * CHUNK
        c1_base = c0_base + CHUNK
        n0_base = jnp.minimum(c0_base + 2 * CHUNK, LAST_BASE)

        cps1 = gathers(c1_base, u1, v1, sem_u1, sem_v1)

        @pl.when(i > 0)
        def _():
            reclaim(out0, sem_o0)

        compute(u0, v0, out0)
        writeback(c0_base, out0, sem_o0)
        for cp in cps1:
            cp.wait()

        cps0 = gathers(n0_base, u0, v0, sem_u0, sem_v0)

        @pl.when(i > 0)
        def _():
            reclaim(out1, sem_o1)

        compute(u1, v1, out1)
        writeback(c1_base, out1, sem_o1)
        for cp in cps0:
            cp.wait()
        return carry

    lax.fori_loop(0, N_PAIRS, pair_body, 0)

    reclaim(out0, sem_o0)
    compute(u0, v0, out0)
    cp = writeback(LAST_BASE, out0, sem_o0)
    reclaim(out1, sem_o1)
    cp.wait()


def kernel(h, edge_index, r):
    hr = _scale_h_by_r(h, r)
    return _edge_scores(hr, h, edge_index.reshape(-1))


# speedup vs baseline: 1.4351x; 1.4351x over previous
import functools

import jax
import jax.numpy as jnp
from jax import lax
from jax.experimental import pallas as pl
from jax.experimental.pallas import tpu as pltpu
from jax.experimental.pallas import tpu_sc as plsc

N_NODES = 10000
N_FEAT = 128
N_EDGES = 320000

NUM_CORES = 2
NUM_SUBCORES = 16
NUM_WORKERS = NUM_CORES * NUM_SUBCORES
EDGES_PER_WORKER = N_EDGES // NUM_WORKERS
CHUNK = 80
N_CHUNKS = EDGES_PER_WORKER // CHUNK
N_PAIRS = N_CHUNKS // 2
LAST_BASE = EDGES_PER_WORKER - CHUNK


def _scale_body(h_ref, r_ref, o_ref):
    o_ref[...] = h_ref[...] * r_ref[...]


def _scale_h_by_r(h, r):
    return pl.pallas_call(
        _scale_body,
        out_shape=jax.ShapeDtypeStruct((N_NODES, N_FEAT), jnp.float32),
    )(h, r.reshape(1, N_FEAT))


_mesh = plsc.VectorSubcoreMesh(core_axis_name="c", subcore_axis_name="s")


@functools.partial(
    pl.kernel,
    mesh=_mesh,
    out_type=jax.ShapeDtypeStruct((N_EDGES,), jnp.float32),
    scratch_types=[
        pltpu.VMEM((CHUNK,), jnp.int32),
        pltpu.VMEM((CHUNK,), jnp.int32),
        pltpu.VMEM((CHUNK,), jnp.int32),
        pltpu.VMEM((CHUNK,), jnp.int32),
        pltpu.VMEM((CHUNK, N_FEAT), jnp.float32),
        pltpu.VMEM((CHUNK, N_FEAT), jnp.float32),
        pltpu.VMEM((CHUNK, N_FEAT), jnp.float32),
        pltpu.VMEM((CHUNK, N_FEAT), jnp.float32),
        pltpu.VMEM((CHUNK,), jnp.float32),
        pltpu.VMEM((CHUNK,), jnp.float32),
        pltpu.SemaphoreType.DMA,
        pltpu.SemaphoreType.DMA,
        pltpu.SemaphoreType.DMA,
        pltpu.SemaphoreType.DMA,
        pltpu.SemaphoreType.DMA,
        pltpu.SemaphoreType.DMA,
        pltpu.SemaphoreType.DMA,
        pltpu.SemaphoreType.DMA,
        pltpu.VMEM_SHARED((N_NODES, N_FEAT), jnp.float32),
    ],
)
def _edge_scores(hr_hbm, h_hbm, ei_hbm, out_hbm,
                 iu0, iu1, iv0, iv1, u0, u1, v0, v1, out0, out1,
                 sem_u0, sem_u1, sem_v0, sem_v1, sem_o0, sem_o1,
                 sem_i0, sem_i1, hr_sp):
    sid = lax.axis_index("s")
    wid = sid * NUM_CORES + lax.axis_index("c")
    base0 = wid * EDGES_PER_WORKER
    lanes = lax.iota(jnp.int32, 16)

    @pl.when(sid < 15)
    def _():
        pltpu.sync_copy(hr_hbm.at[pl.ds(sid * 640, 640), :],
                        hr_sp.at[pl.ds(sid * 640, 640), :])

    @pl.when(sid == 15)
    def _():
        pltpu.sync_copy(hr_hbm.at[pl.ds(9600, 400), :],
                        hr_sp.at[pl.ds(9600, 400), :])

    plsc.subcore_barrier()

    def fetch_idx(base, iu, iv, sem):
        pltpu.async_copy(ei_hbm.at[pl.ds(base0 + base, CHUNK)], iu, sem)
        pltpu.async_copy(ei_hbm.at[pl.ds(N_EDGES + base0 + base, CHUNK)], iv, sem)

    def drain_idx(iu, iv, sem):
        pltpu.make_async_copy(ei_hbm.at[pl.ds(base0, CHUNK)], iu, sem).wait()
        pltpu.make_async_copy(ei_hbm.at[pl.ds(base0, CHUNK)], iv, sem).wait()

    def gathers(base_unused, iu, iv, u_buf, v_buf, su, sv):
        cu = pltpu.async_copy(hr_sp.at[iu], u_buf, su)
        cv = pltpu.async_copy(h_hbm.at[iv], v_buf, sv)
        return cu, cv

    msk8 = lanes < 8
    msk4 = (lanes & 4) == 0
    quad_pat = (lanes & 1) * 8 + ((lanes >> 1) & 1) * 4

    def step(v, sh):
        return v + jnp.take(v, lanes ^ sh)

    def compute(u_buf, v_buf, out_buf):
        def quad_body(k, scores):
            q = k & 3
            accs = []
            for t in range(4):
                e = k * 4 + t
                p = [u_buf[e, pl.ds(16 * j, 16)] * v_buf[e, pl.ds(16 * j, 16)]
                     for j in range(N_FEAT // 16)]
                accs.append(((p[0] + p[1]) + (p[2] + p[3]))
                            + ((p[4] + p[5]) + (p[6] + p[7])))
            m0 = jnp.where(msk8, step(accs[0], 8), step(accs[1], 8))
            m1 = jnp.where(msk8, step(accs[2], 8), step(accs[3], 8))
            n = jnp.where(msk4, step(m0, 4), step(m1, 4))
            full = step(step(n, 2), 1)
            scores = jnp.where(q == 0, jnp.zeros((16,), jnp.float32), scores)
            scores = jnp.where((lanes >> 2) == q, jnp.take(full, quad_pat), scores)
            out_buf[pl.ds((k >> 2) * 16, 16)] = scores
            return scores

        lax.fori_loop(0, CHUNK // 4, quad_body, jnp.zeros((16,), jnp.float32))

    def writeback(base, out_buf, sem):
        return pltpu.async_copy(out_buf, out_hbm.at[pl.ds(base0 + base, CHUNK)], sem)

    def reclaim(out_buf, sem):
        pltpu.make_async_copy(out_buf, out_hbm.at[pl.ds(base0, CHUNK)], sem).wait()

    fetch_idx(0, iu0, iv0, sem_i0)
    drain_idx(iu0, iv0, sem_i0)
    cu, cv = gathers(0, iu0, iv0, u0, v0, sem_u0, sem_v0)
    fetch_idx(CHUNK, iu1, iv1, sem_i1)
    cu.wait()
    cv.wait()

    def pair_body(i, carry):
        c0_base = (2 * i) * CHUNK
        c1_base = c0_base + CHUNK
        n0_base = jnp.minimum(c0_base + 2 * CHUNK, LAST_BASE)
        n1_base = jnp.minimum(c0_base + 3 * CHUNK, LAST_BASE)

        drain_idx(iu1, iv1, sem_i1)
        cu1, cv1 = gathers(c1_base, iu1, iv1, u1, v1, sem_u1, sem_v1)
        fetch_idx(n0_base, iu0, iv0, sem_i0)

        @pl.when(i > 0)
        def _():
            reclaim(out0, sem_o0)

        compute(u0, v0, out0)
        writeback(c0_base, out0, sem_o0)
        cu1.wait()
        cv1.wait()

        drain_idx(iu0, iv0, sem_i0)
        cu0, cv0 = gathers(n0_base, iu0, iv0, u0, v0, sem_u0, sem_v0)
        fetch_idx(n1_base, iu1, iv1, sem_i1)

        @pl.when(i > 0)
        def _():
            reclaim(out1, sem_o1)

        compute(u1, v1, out1)
        writeback(c1_base, out1, sem_o1)
        cu0.wait()
        cv0.wait()
        return carry

    lax.fori_loop(0, N_PAIRS, pair_body, 0)

    drain_idx(iu1, iv1, sem_i1)
    reclaim(out0, sem_o0)
    compute(u0, v0, out0)
    cp = writeback(LAST_BASE, out0, sem_o0)
    reclaim(out1, sem_o1)
    cp.wait()


def kernel(h, edge_index, r):
    hr = _scale_h_by_r(h, r)
    return _edge_scores(hr, h, edge_index.reshape(-1))


# speedup vs baseline: 1.4465x; 1.0080x over previous
import functools

import jax
import jax.numpy as jnp
from jax import lax
from jax.experimental import pallas as pl
from jax.experimental.pallas import tpu as pltpu
from jax.experimental.pallas import tpu_sc as plsc

N_NODES = 10000
N_FEAT = 128
N_EDGES = 320000

NUM_CORES = 2
NUM_SUBCORES = 16
NUM_WORKERS = NUM_CORES * NUM_SUBCORES
EDGES_PER_WORKER = N_EDGES // NUM_WORKERS
CHUNK = 96
N_CHUNKS = -(-EDGES_PER_WORKER // CHUNK)
N_PAIRS = N_CHUNKS // 2
LAST_BASE = EDGES_PER_WORKER - CHUNK


def _scale_body(h_ref, r_ref, o_ref):
    o_ref[...] = h_ref[...] * r_ref[...]


def _scale_h_by_r(h, r):
    return pl.pallas_call(
        _scale_body,
        out_shape=jax.ShapeDtypeStruct((N_NODES, N_FEAT), jnp.float32),
    )(h, r.reshape(1, N_FEAT))


_mesh = plsc.VectorSubcoreMesh(core_axis_name="c", subcore_axis_name="s")


@functools.partial(
    pl.kernel,
    mesh=_mesh,
    out_type=jax.ShapeDtypeStruct((N_EDGES,), jnp.float32),
    scratch_types=[
        pltpu.VMEM((CHUNK,), jnp.int32),
        pltpu.VMEM((CHUNK,), jnp.int32),
        pltpu.VMEM((CHUNK,), jnp.int32),
        pltpu.VMEM((CHUNK,), jnp.int32),
        pltpu.VMEM((CHUNK, N_FEAT), jnp.float32),
        pltpu.VMEM((CHUNK, N_FEAT), jnp.float32),
        pltpu.VMEM((CHUNK, N_FEAT), jnp.float32),
        pltpu.VMEM((CHUNK, N_FEAT), jnp.float32),
        pltpu.VMEM((CHUNK,), jnp.float32),
        pltpu.VMEM((CHUNK,), jnp.float32),
        pltpu.SemaphoreType.DMA,
        pltpu.SemaphoreType.DMA,
        pltpu.SemaphoreType.DMA,
        pltpu.SemaphoreType.DMA,
        pltpu.SemaphoreType.DMA,
        pltpu.SemaphoreType.DMA,
        pltpu.SemaphoreType.DMA,
        pltpu.SemaphoreType.DMA,
        pltpu.VMEM_SHARED((N_NODES, N_FEAT), jnp.float32),
    ],
)
def _edge_scores(hr_hbm, h_hbm, ei_hbm, out_hbm,
                 iu0, iu1, iv0, iv1, u0, u1, v0, v1, out0, out1,
                 sem_u0, sem_u1, sem_v0, sem_v1, sem_o0, sem_o1,
                 sem_i0, sem_i1, hr_sp):
    sid = lax.axis_index("s")
    wid = sid * NUM_CORES + lax.axis_index("c")
    base0 = wid * EDGES_PER_WORKER
    lanes = lax.iota(jnp.int32, 16)

    @pl.when(sid < 15)
    def _():
        pltpu.sync_copy(hr_hbm.at[pl.ds(sid * 640, 640), :],
                        hr_sp.at[pl.ds(sid * 640, 640), :])

    @pl.when(sid == 15)
    def _():
        pltpu.sync_copy(hr_hbm.at[pl.ds(9600, 400), :],
                        hr_sp.at[pl.ds(9600, 400), :])

    plsc.subcore_barrier()

    def fetch_idx(base, iu, iv, sem):
        pltpu.async_copy(ei_hbm.at[pl.ds(base0 + base, CHUNK)], iu, sem)
        pltpu.async_copy(ei_hbm.at[pl.ds(N_EDGES + base0 + base, CHUNK)], iv, sem)

    def drain_idx(iu, iv, sem):
        pltpu.make_async_copy(ei_hbm.at[pl.ds(base0, CHUNK)], iu, sem).wait()
        pltpu.make_async_copy(ei_hbm.at[pl.ds(base0, CHUNK)], iv, sem).wait()

    def gathers(base_unused, iu, iv, u_buf, v_buf, su, sv):
        cu = pltpu.async_copy(hr_sp.at[iu], u_buf, su)
        cv = pltpu.async_copy(h_hbm.at[iv], v_buf, sv)
        return cu, cv

    msk8 = lanes < 8
    msk4 = (lanes & 4) == 0
    quad_pat = (lanes & 1) * 8 + ((lanes >> 1) & 1) * 4

    def step(v, sh):
        return v + jnp.take(v, lanes ^ sh)

    def compute(u_buf, v_buf, out_buf):
        def quad_body(k, scores):
            q = k & 3
            accs = []
            for t in range(4):
                e = k * 4 + t
                p = [u_buf[e, pl.ds(16 * j, 16)] * v_buf[e, pl.ds(16 * j, 16)]
                     for j in range(N_FEAT // 16)]
                accs.append(((p[0] + p[1]) + (p[2] + p[3]))
                            + ((p[4] + p[5]) + (p[6] + p[7])))
            m0 = jnp.where(msk8, step(accs[0], 8), step(accs[1], 8))
            m1 = jnp.where(msk8, step(accs[2], 8), step(accs[3], 8))
            n = jnp.where(msk4, step(m0, 4), step(m1, 4))
            full = step(step(n, 2), 1)
            scores = jnp.where(q == 0, jnp.zeros((16,), jnp.float32), scores)
            scores = jnp.where((lanes >> 2) == q, jnp.take(full, quad_pat), scores)
            out_buf[pl.ds((k >> 2) * 16, 16)] = scores
            return scores

        lax.fori_loop(0, CHUNK // 4, quad_body, jnp.zeros((16,), jnp.float32))

    def writeback(base, out_buf, sem):
        return pltpu.async_copy(out_buf, out_hbm.at[pl.ds(base0 + base, CHUNK)], sem)

    def reclaim(out_buf, sem):
        pltpu.make_async_copy(out_buf, out_hbm.at[pl.ds(base0, CHUNK)], sem).wait()

    fetch_idx(0, iu0, iv0, sem_i0)
    drain_idx(iu0, iv0, sem_i0)
    cu, cv = gathers(0, iu0, iv0, u0, v0, sem_u0, sem_v0)
    fetch_idx(CHUNK, iu1, iv1, sem_i1)
    cu.wait()
    cv.wait()

    def pair_body(i, carry):
        c0_base = (2 * i) * CHUNK
        c1_base = c0_base + CHUNK
        n0_base = jnp.minimum(c0_base + 2 * CHUNK, LAST_BASE)
        n1_base = jnp.minimum(c0_base + 3 * CHUNK, LAST_BASE)

        drain_idx(iu1, iv1, sem_i1)
        cu1, cv1 = gathers(c1_base, iu1, iv1, u1, v1, sem_u1, sem_v1)
        fetch_idx(n0_base, iu0, iv0, sem_i0)

        @pl.when(i > 0)
        def _():
            reclaim(out0, sem_o0)

        compute(u0, v0, out0)
        writeback(c0_base, out0, sem_o0)
        cu1.wait()
        cv1.wait()

        drain_idx(iu0, iv0, sem_i0)
        cu0, cv0 = gathers(n0_base, iu0, iv0, u0, v0, sem_u0, sem_v0)
        fetch_idx(n1_base, iu1, iv1, sem_i1)

        @pl.when(i > 0)
        def _():
            reclaim(out1, sem_o1)

        compute(u1, v1, out1)
        writeback(c1_base, out1, sem_o1)
        cu0.wait()
        cv0.wait()
        return carry

    lax.fori_loop(0, N_PAIRS, pair_body, 0)

    drain_idx(iu1, iv1, sem_i1)
    reclaim(out0, sem_o0)
    compute(u0, v0, out0)
    cp = writeback(LAST_BASE, out0, sem_o0)
    reclaim(out1, sem_o1)
    cp.wait()


def kernel(h, edge_index, r):
    hr = _scale_h_by_r(h, r)
    return _edge_scores(hr, h, edge_index.reshape(-1))


# speedup vs baseline: 1.4470x; 1.0003x over previous
import functools

import jax
import jax.numpy as jnp
from jax import lax
from jax.experimental import pallas as pl
from jax.experimental.pallas import tpu as pltpu
from jax.experimental.pallas import tpu_sc as plsc

N_NODES = 10000
N_FEAT = 128
N_EDGES = 320000

NUM_CORES = 2
NUM_SUBCORES = 16
NUM_WORKERS = NUM_CORES * NUM_SUBCORES
EDGES_PER_WORKER = N_EDGES // NUM_WORKERS
CHUNK = 96
N_CHUNKS = -(-EDGES_PER_WORKER // CHUNK)
N_PAIRS = N_CHUNKS // 2
LAST_BASE = EDGES_PER_WORKER - CHUNK


def _scale_body(h_ref, r_ref, o_ref):
    o_ref[...] = h_ref[...] * r_ref[...]


def _scale_h_by_r(h, r):
    return pl.pallas_call(
        _scale_body,
        out_shape=jax.ShapeDtypeStruct((N_NODES, N_FEAT), jnp.float32),
    )(h, r.reshape(1, N_FEAT))


_mesh = plsc.VectorSubcoreMesh(core_axis_name="c", subcore_axis_name="s")


@functools.partial(
    pl.kernel,
    mesh=_mesh,
    out_type=jax.ShapeDtypeStruct((N_EDGES,), jnp.float32),
    scratch_types=[
        pltpu.VMEM((CHUNK,), jnp.int32),
        pltpu.VMEM((CHUNK,), jnp.int32),
        pltpu.VMEM((CHUNK,), jnp.int32),
        pltpu.VMEM((CHUNK,), jnp.int32),
        pltpu.VMEM((CHUNK, N_FEAT), jnp.float32),
        pltpu.VMEM((CHUNK, N_FEAT), jnp.float32),
        pltpu.VMEM((CHUNK, N_FEAT), jnp.float32),
        pltpu.VMEM((CHUNK, N_FEAT), jnp.float32),
        pltpu.VMEM((CHUNK,), jnp.float32),
        pltpu.VMEM((CHUNK,), jnp.float32),
        pltpu.SemaphoreType.DMA,
        pltpu.SemaphoreType.DMA,
        pltpu.SemaphoreType.DMA,
        pltpu.SemaphoreType.DMA,
        pltpu.SemaphoreType.DMA,
        pltpu.SemaphoreType.DMA,
        pltpu.SemaphoreType.DMA,
        pltpu.SemaphoreType.DMA,
        pltpu.VMEM_SHARED((N_NODES, N_FEAT), jnp.float32),
    ],
)
def _edge_scores(hr_hbm, h_hbm, ei_hbm, out_hbm,
                 iu0, iu1, iv0, iv1, u0, u1, v0, v1, out0, out1,
                 sem_u0, sem_u1, sem_v0, sem_v1, sem_o0, sem_o1,
                 sem_i0, sem_i1, hr_sp):
    sid = lax.axis_index("s")
    wid = sid * NUM_CORES + lax.axis_index("c")
    base0 = wid * EDGES_PER_WORKER
    lanes = lax.iota(jnp.int32, 16)

    @pl.when(sid < 15)
    def _():
        pltpu.sync_copy(hr_hbm.at[pl.ds(sid * 640, 640), :],
                        hr_sp.at[pl.ds(sid * 640, 640), :])

    @pl.when(sid == 15)
    def _():
        pltpu.sync_copy(hr_hbm.at[pl.ds(9600, 400), :],
                        hr_sp.at[pl.ds(9600, 400), :])

    plsc.subcore_barrier()

    def fetch_idx(base, iu, iv, sem):
        pltpu.async_copy(ei_hbm.at[pl.ds(base0 + base, CHUNK)], iu, sem)
        pltpu.async_copy(ei_hbm.at[pl.ds(N_EDGES + base0 + base, CHUNK)], iv, sem)

    def drain_idx(iu, iv, sem):
        pltpu.make_async_copy(ei_hbm.at[pl.ds(base0, CHUNK)], iu, sem).wait()
        pltpu.make_async_copy(ei_hbm.at[pl.ds(base0, CHUNK)], iv, sem).wait()

    def gathers(iu, iv, u_buf, v_buf, su, sv):
        cu = pltpu.async_copy(hr_sp.at[iu], u_buf, su)
        cv = pltpu.async_copy(h_hbm.at[iv], v_buf, sv)
        return cu, cv

    msk8 = lanes < 8
    msk4 = (lanes & 4) == 0
    quad_pat = (lanes & 1) * 8 + ((lanes >> 1) & 1) * 4

    def step(v, sh):
        return v + jnp.take(v, lanes ^ sh)

    def compute(u_buf, v_buf, out_buf):
        def quad_body(k, scores):
            q = k & 3
            accs = []
            for t in range(4):
                e = k * 4 + t
                p = [u_buf[e, pl.ds(16 * j, 16)] * v_buf[e, pl.ds(16 * j, 16)]
                     for j in range(N_FEAT // 16)]
                accs.append(((p[0] + p[1]) + (p[2] + p[3]))
                            + ((p[4] + p[5]) + (p[6] + p[7])))
            m0 = jnp.where(msk8, step(accs[0], 8), step(accs[1], 8))
            m1 = jnp.where(msk8, step(accs[2], 8), step(accs[3], 8))
            n = jnp.where(msk4, step(m0, 4), step(m1, 4))
            full = step(step(n, 2), 1)
            scores = jnp.where(q == 0, jnp.zeros((16,), jnp.float32), scores)
            scores = jnp.where((lanes >> 2) == q, jnp.take(full, quad_pat), scores)
            out_buf[pl.ds((k >> 2) * 16, 16)] = scores
            return scores

        lax.fori_loop(0, CHUNK // 4, quad_body, jnp.zeros((16,), jnp.float32))

    def writeback(base, out_buf, sem):
        return pltpu.async_copy(out_buf, out_hbm.at[pl.ds(base0 + base, CHUNK)], sem)

    def reclaim(out_buf, sem):
        pltpu.make_async_copy(out_buf, out_hbm.at[pl.ds(base0, CHUNK)], sem).wait()

    fetch_idx(0, iu0, iv0, sem_i0)
    drain_idx(iu0, iv0, sem_i0)
    cu, cv = gathers(iu0, iv0, u0, v0, sem_u0, sem_v0)
    fetch_idx(CHUNK, iu1, iv1, sem_i1)
    cu.wait()
    cv.wait()

    def pair_body(i, carry):
        c0_base = (2 * i) * CHUNK
        c1_base = c0_base + CHUNK
        n0_base = jnp.minimum(c0_base + 2 * CHUNK, LAST_BASE)
        n1_base = jnp.minimum(c0_base + 3 * CHUNK, LAST_BASE)

        drain_idx(iu1, iv1, sem_i1)
        cu1, cv1 = gathers(iu1, iv1, u1, v1, sem_u1, sem_v1)
        fetch_idx(n0_base, iu0, iv0, sem_i0)

        @pl.when(i > 0)
        def _():
            reclaim(out0, sem_o0)

        compute(u0, v0, out0)
        writeback(c0_base, out0, sem_o0)
        cu1.wait()
        cv1.wait()

        drain_idx(iu0, iv0, sem_i0)
        cu0, cv0 = gathers(iu0, iv0, u0, v0, sem_u0, sem_v0)
        fetch_idx(n1_base, iu1, iv1, sem_i1)

        @pl.when(i > 0)
        def _():
            reclaim(out1, sem_o1)

        compute(u1, v1, out1)
        writeback(c1_base, out1, sem_o1)
        cu0.wait()
        cv0.wait()
        return carry

    lax.fori_loop(0, N_PAIRS, pair_body, 0)

    drain_idx(iu1, iv1, sem_i1)
    reclaim(out0, sem_o0)
    compute(u0, v0, out0)
    cp = writeback(LAST_BASE, out0, sem_o0)
    reclaim(out1, sem_o1)
    cp.wait()


def kernel(h, edge_index, r):
    hr = _scale_h_by_r(h, r)
    return _edge_scores(hr, h, edge_index.reshape(-1))
